# baseline (speedup 1.0000x reference)
import functools

import jax
import jax.numpy as jnp
from jax import lax
from jax.experimental import pallas as pl
from jax.experimental.pallas import tpu as pltpu

B = 1024
T = 4096
FFN = 2048
HID = 8192

HBLK = 256
NH = HID // HBLK


def _mlp_body(x_ref, w1_ref, w3_ref, w2_ref, rw_ref, out_ref):
    j = pl.program_id(0)
    x = x_ref[...]
    w1 = w1_ref[...]
    w3 = w3_ref[...]
    h1 = lax.dot_general(x, w1, (((1,), (1,)), ((), ())),
                         preferred_element_type=jnp.float32)
    h3 = lax.dot_general(x, w3, (((1,), (1,)), ((), ())),
                         preferred_element_type=jnp.float32)
    g = (h1 * jax.nn.sigmoid(h1)) * h3
    w2 = w2_ref[...]
    contrib = lax.dot_general(g, w2,
                              (((1,), (1,)), ((), ())),
                              preferred_element_type=jnp.float32)

    @pl.when(j == 0)
    def _():
        out_ref[...] = contrib

    @pl.when(j > 0)
    def _():
        out_ref[...] += contrib

    @pl.when(j == NH - 1)
    def _():
        out_ref[...] *= rw_ref[...]


def _mlp(x, routing_weights, W1, W3, W2, interpret=False):
    return pl.pallas_call(
        _mlp_body,
        grid=(NH,),
        in_specs=[
            pl.BlockSpec((B, FFN), lambda j: (0, 0)),
            pl.BlockSpec((HBLK, FFN), lambda j: (j, 0)),
            pl.BlockSpec((HBLK, FFN), lambda j: (j, 0)),
            pl.BlockSpec((FFN, HBLK), lambda j: (0, j)),
            pl.BlockSpec((B, 1), lambda j: (0, 0)),
        ],
        out_specs=pl.BlockSpec((B, FFN), lambda j: (0, 0)),
        out_shape=jax.ShapeDtypeStruct((B, FFN), jnp.float32),
        compiler_params=pltpu.CompilerParams(
            dimension_semantics=("arbitrary",),
        ),
        interpret=interpret,
    )(x, W1, W3, W2, routing_weights)


def kernel(hidden_states, input_idx, routing_weights, W1, W2, W3):
    x = jnp.take(hidden_states, input_idx, axis=0)
    return _mlp(x, routing_weights, W1, W3, W2)


# speedup vs baseline: 1.0792x; 1.0792x over previous
import functools

import jax
import jax.numpy as jnp
from jax import lax
from jax.experimental import pallas as pl
from jax.experimental.pallas import tpu as pltpu

B = 1024
T = 4096
FFN = 2048
HID = 8192

HBLK = 512
NH = HID // HBLK


def _mlp_body(x_ref, w1_ref, w3_ref, w2_ref, rw_ref, out_ref):
    j = pl.program_id(0)
    x = x_ref[...]
    w1 = w1_ref[...]
    w3 = w3_ref[...]
    h1 = lax.dot_general(x, w1, (((1,), (1,)), ((), ())),
                         preferred_element_type=jnp.float32)
    h3 = lax.dot_general(x, w3, (((1,), (1,)), ((), ())),
                         preferred_element_type=jnp.float32)
    g = (h1 * jax.nn.sigmoid(h1)) * h3 * rw_ref[...]
    w2 = w2_ref[...]
    contrib = lax.dot_general(g, w2,
                              (((1,), (1,)), ((), ())),
                              preferred_element_type=jnp.float32)

    @pl.when(j == 0)
    def _():
        out_ref[...] = contrib

    @pl.when(j > 0)
    def _():
        out_ref[...] += contrib


def _mlp(x, routing_weights, W1, W3, W2, interpret=False):
    return pl.pallas_call(
        _mlp_body,
        grid=(NH,),
        in_specs=[
            pl.BlockSpec((B, FFN), lambda j: (0, 0)),
            pl.BlockSpec((HBLK, FFN), lambda j: (j, 0)),
            pl.BlockSpec((HBLK, FFN), lambda j: (j, 0)),
            pl.BlockSpec((FFN, HBLK), lambda j: (0, j)),
            pl.BlockSpec((B, 1), lambda j: (0, 0)),
        ],
        out_specs=pl.BlockSpec((B, FFN), lambda j: (0, 0)),
        out_shape=jax.ShapeDtypeStruct((B, FFN), jnp.float32),
        compiler_params=pltpu.CompilerParams(
            dimension_semantics=("arbitrary",),
        ),
        interpret=interpret,
    )(x, W1, W3, W2, routing_weights)


def kernel(hidden_states, input_idx, routing_weights, W1, W2, W3):
    x = jnp.take(hidden_states, input_idx, axis=0)
    return _mlp(x, routing_weights, W1, W3, W2)


# speedup vs baseline: 1.0863x; 1.0065x over previous
import functools

import jax
import jax.numpy as jnp
from jax import lax
from jax.experimental import pallas as pl
from jax.experimental.pallas import tpu as pltpu

B = 1024
T = 4096
FFN = 2048
HID = 8192

HBLK = 512
NH = HID // HBLK


def _mlp_body(x_ref, w1_ref, w3_ref, w2_ref, rw_ref, out_ref):
    j = pl.program_id(0)
    x = x_ref[...].astype(jnp.bfloat16)
    w1 = w1_ref[...].astype(jnp.bfloat16)
    w3 = w3_ref[...].astype(jnp.bfloat16)
    h1 = lax.dot_general(x, w1, (((1,), (1,)), ((), ())),
                         preferred_element_type=jnp.float32)
    h3 = lax.dot_general(x, w3, (((1,), (1,)), ((), ())),
                         preferred_element_type=jnp.float32)
    g = (h1 * jax.nn.sigmoid(h1)) * h3 * rw_ref[...]
    w2 = w2_ref[...].astype(jnp.bfloat16)
    contrib = lax.dot_general(g.astype(jnp.bfloat16), w2,
                              (((1,), (1,)), ((), ())),
                              preferred_element_type=jnp.float32)

    @pl.when(j == 0)
    def _():
        out_ref[...] = contrib

    @pl.when(j > 0)
    def _():
        out_ref[...] += contrib


def _mlp(x, routing_weights, W1, W3, W2, interpret=False):
    return pl.pallas_call(
        _mlp_body,
        grid=(NH,),
        in_specs=[
            pl.BlockSpec((B, FFN), lambda j: (0, 0)),
            pl.BlockSpec((HBLK, FFN), lambda j: (j, 0)),
            pl.BlockSpec((HBLK, FFN), lambda j: (j, 0)),
            pl.BlockSpec((FFN, HBLK), lambda j: (0, j)),
            pl.BlockSpec((B, 1), lambda j: (0, 0)),
        ],
        out_specs=pl.BlockSpec((B, FFN), lambda j: (0, 0)),
        out_shape=jax.ShapeDtypeStruct((B, FFN), jnp.float32),
        compiler_params=pltpu.CompilerParams(
            dimension_semantics=("arbitrary",),
        ),
        interpret=interpret,
    )(x, W1, W3, W2, routing_weights)


def kernel(hidden_states, input_idx, routing_weights, W1, W2, W3):
    x = jnp.take(hidden_states, input_idx, axis=0)
    return _mlp(x, routing_weights, W1, W3, W2)


# speedup vs baseline: 1.1028x; 1.0152x over previous
import jax
import jax.numpy as jnp
from jax import lax
from jax.experimental import pallas as pl
from jax.experimental.pallas import tpu as pltpu

B = 1024
T = 4096
FFN = 2048
HID = 8192

HBLK = 512
NH = HID // HBLK
FBLK = 256
NF = FFN // FBLK


def _gate_body(x_ref, w1_ref, w3_ref, rw_ref, g_ref):
    x = x_ref[...].astype(jnp.bfloat16)
    w1 = w1_ref[...].astype(jnp.bfloat16)
    w3 = w3_ref[...].astype(jnp.bfloat16)
    h1 = lax.dot_general(x, w1, (((1,), (1,)), ((), ())),
                         preferred_element_type=jnp.float32)
    h3 = lax.dot_general(x, w3, (((1,), (1,)), ((), ())),
                         preferred_element_type=jnp.float32)
    g = (h1 * jax.nn.sigmoid(h1)) * h3 * rw_ref[...]
    g_ref[...] = g.astype(jnp.bfloat16)


def _gate(x, routing_weights, W1, W3):
    return pl.pallas_call(
        _gate_body,
        grid=(NH,),
        in_specs=[pl.BlockSpec((B, FFN), lambda j: (0, 0)),
                  pl.BlockSpec((HBLK, FFN), lambda j: (j, 0)),
                  pl.BlockSpec((HBLK, FFN), lambda j: (j, 0)),
                  pl.BlockSpec((B, 1), lambda j: (0, 0))],
        out_specs=pl.BlockSpec((B, HBLK), lambda j: (0, j)),
        out_shape=jax.ShapeDtypeStruct((B, HID), jnp.bfloat16),
        compiler_params=pltpu.CompilerParams(
            dimension_semantics=("arbitrary",)),
    )(x, W1, W3, routing_weights)


def _down_body(g_ref, w2_ref, o_ref):
    w2 = w2_ref[...].astype(jnp.bfloat16)
    o_ref[...] = lax.dot_general(g_ref[...], w2, (((1,), (1,)), ((), ())),
                                 preferred_element_type=jnp.float32)


def _down(g, W2):
    return pl.pallas_call(
        _down_body,
        grid=(NF,),
        in_specs=[pl.BlockSpec((B, HID), lambda j: (0, 0)),
                  pl.BlockSpec((FBLK, HID), lambda j: (j, 0))],
        out_specs=pl.BlockSpec((B, FBLK), lambda j: (0, j)),
        out_shape=jax.ShapeDtypeStruct((B, FFN), jnp.float32),
        compiler_params=pltpu.CompilerParams(
            dimension_semantics=("arbitrary",)),
    )(g, W2)


def kernel(hidden_states, input_idx, routing_weights, W1, W2, W3):
    x = jnp.take(hidden_states, input_idx, axis=0)
    g = _gate(x, routing_weights, W1, W3)
    return _down(g, W2)


# speedup vs baseline: 1.2193x; 1.1056x over previous
import jax
import jax.numpy as jnp
from jax import lax
from jax.experimental import pallas as pl
from jax.experimental.pallas import tpu as pltpu

B = 1024
T = 4096
FFN = 2048
HID = 8192

HBLK = 256
NH = HID // HBLK
FBLK = 256
NF = FFN // FBLK
GCHUNK = 256


def _body(idx_ref, hs_ref, w1_ref, w3_ref, w2_ref, rw_ref, o_ref,
          x_ref, g_ref, sem):
    j = pl.program_id(0)

    @pl.when(j == 0)
    def _gather():
        def issue(i, _):
            pltpu.make_async_copy(hs_ref.at[idx_ref[i]], x_ref.at[i], sem).start()
            return 0

        def drain(i, _):
            pltpu.make_async_copy(hs_ref.at[0], x_ref.at[0], sem).wait()
            return 0

        def per_chunk(c, _):
            lax.fori_loop(c * GCHUNK, (c + 1) * GCHUNK, issue, 0)
            lax.fori_loop(0, GCHUNK, drain, 0)
            return 0

        lax.fori_loop(0, B // GCHUNK, per_chunk, 0)

    @pl.when(j < NH)
    def _gate():
        x = x_ref[...].astype(jnp.bfloat16)
        w1 = w1_ref[...].astype(jnp.bfloat16)
        w3 = w3_ref[...].astype(jnp.bfloat16)
        h1 = lax.dot_general(x, w1, (((1,), (1,)), ((), ())),
                             preferred_element_type=jnp.float32)
        h3 = lax.dot_general(x, w3, (((1,), (1,)), ((), ())),
                             preferred_element_type=jnp.float32)
        g = (h1 * jax.nn.sigmoid(h1)) * h3 * rw_ref[...]
        g_ref[:, pl.ds(j * HBLK, HBLK)] = g.astype(jnp.bfloat16)

    @pl.when(j >= NH)
    def _down():
        w2 = w2_ref[...].astype(jnp.bfloat16)
        o_ref[...] = lax.dot_general(g_ref[...], w2, (((1,), (1,)), ((), ())),
                                     preferred_element_type=jnp.float32)


def kernel(hidden_states, input_idx, routing_weights, W1, W2, W3):
    return pl.pallas_call(
        _body,
        grid=(NH + NF,),
        in_specs=[
            pl.BlockSpec(memory_space=pltpu.SMEM),
            pl.BlockSpec(memory_space=pl.ANY),
            pl.BlockSpec((HBLK, FFN), lambda j: (jnp.minimum(j, NH - 1), 0)),
            pl.BlockSpec((HBLK, FFN), lambda j: (jnp.minimum(j, NH - 1), 0)),
            pl.BlockSpec((FBLK, HID), lambda j: (jnp.maximum(j - NH, 0), 0)),
            pl.BlockSpec((B, 1), lambda j: (0, 0)),
        ],
        out_specs=pl.BlockSpec((B, FBLK), lambda j: (0, jnp.maximum(j - NH, 0))),
        out_shape=jax.ShapeDtypeStruct((B, FFN), jnp.float32),
        scratch_shapes=[
            pltpu.VMEM((B, FFN), jnp.float32),
            pltpu.VMEM((B, HID), jnp.bfloat16),
            pltpu.SemaphoreType.DMA,
        ],
        compiler_params=pltpu.CompilerParams(
            dimension_semantics=("arbitrary",)),
    )(input_idx, hidden_states, W1, W3, W2, routing_weights)
